# Initial kernel scaffold; baseline (speedup 1.0000x reference)
import jax
import jax.numpy as jnp
from jax.experimental import pallas as pl


def kernel(x, edge_index, batch, Wc1, bc1, Wc2, bc2, Wb1, bb1, Wb2, bb2, Mc1, mc1, Mc2, mc2, Mb1, mb1, Mb2, mb2, Wd, bd, Ws, bs, Wa, ba, Wdb, bdb):
    raise NotImplementedError("write your pallas kernel here")


# speedup vs baseline: 5.5897x; 5.5897x over previous
import functools

import jax
import jax.numpy as jnp
from jax import lax
from jax.experimental import pallas as pl
from jax.experimental.pallas import tpu as pltpu
from jax.experimental.pallas import tpu_sc as plsc

N = 10000
E = 320000
G = 64
F = 128
NSUB = 16
RPT = N // NSUB
AW = 80

CHM = 128
NCHUNK_M = E // CHM
ITERS_M = -(-NCHUNK_M // NSUB)

CHL = 64
NCHUNK_L = E // CHL
ITERS_L = -(-NCHUNK_L // NSUB)


@functools.cache
def _mesh():
    return plsc.VectorSubcoreMesh(
        core_axis_name="c", subcore_axis_name="s",
        num_cores=2, num_subcores=NSUB,
    )


_SC_PARAMS = pltpu.CompilerParams(use_tc_tiling_on_sc=False)


def _mask_body(psrc, pdst, eidx, m2t, zc16, wout, deg,
               eb0, eb1, sdid0, sdid1, srows0, srows1, drows0, drows1,
               wbuf0, wbuf1, wpad0, wpad1, m2v, accD,
               isem0, isem1, gsem0, gsem1, hsem0, hsem1, osem0, osem1,
               ssem0, ssem1):
    eb = (eb0, eb1)
    sdid = (sdid0, sdid1)
    srows = (srows0, srows1)
    drows = (drows0, drows1)
    wbuf = (wbuf0, wbuf1)
    wpad = (wpad0, wpad1)
    isem = (isem0, isem1)
    gsem = (gsem0, gsem1)
    hsem = (hsem0, hsem1)
    osem = (osem0, osem1)
    ssem = (ssem0, ssem1)
    cid = lax.axis_index("c")
    sid = lax.axis_index("s")
    pltpu.sync_copy(m2t, m2v)
    off = cid * N
    iota = lax.iota(jnp.int32, 16)
    m2r = [m2v[cid, pl.ds(16 * j, 16)] for j in range(4)]
    bias = m2v[cid, pl.ds(64, 16)][0]
    perms = [iota ^ sh for sh in (1, 2, 4, 8)]

    def cbase(it):
        return (sid + it * NSUB) * CHM

    def issue_idx(it, b):
        pltpu.async_copy(eidx.at[:, pl.ds(cbase(it), CHM)], eb[b], isem[b])

    def wait_idx(it, b):
        pltpu.make_async_copy(
            eidx.at[:, pl.ds(cbase(it), CHM)], eb[b], isem[b]).wait()

    def issue_gather(b):
        for j in range(CHM // 16):
            eb[b][0, pl.ds(16 * j, 16)] = eb[b][0, pl.ds(16 * j, 16)] + off
            eb[b][1, pl.ds(16 * j, 16)] = eb[b][1, pl.ds(16 * j, 16)] + off
        pltpu.async_copy(psrc.at[eb[b].at[0]], srows[b], gsem[b])
        pltpu.async_copy(pdst.at[eb[b].at[1]], drows[b], hsem[b])

    def wait_gather(b):
        pltpu.make_async_copy(psrc.at[eb[b].at[0]], srows[b], gsem[b]).wait()
        pltpu.make_async_copy(pdst.at[eb[b].at[1]], drows[b], hsem[b]).wait()

    def drain_out(b):
        pltpu.make_async_copy(
            wbuf[b], wout.at[cid, pl.ds(0, CHM)], osem[b]).wait()

    def drain_scatter(b):
        pltpu.make_async_copy(wpad[b], accD.at[sdid[b].at[0]],
                              ssem[b]).wait()

    r0 = sid * RPT
    pltpu.sync_copy(zc16.at[pl.ds(r0, RPT)], accD.at[pl.ds(r0, RPT)])
    issue_idx(0, 0)
    issue_idx(1, 1)
    wait_idx(0, 0)
    issue_gather(0)
    plsc.subcore_barrier()

    def outer(it2, carry):
        for u in range(2):
            it = it2 * 2 + u
            b = u
            c = sid + it * NSUB

            @pl.when(c < NCHUNK_M)
            def _():
                @pl.when(it >= 2)
                def _():
                    drain_out(b)
                    drain_scatter(b)

                wait_gather(b)

                @pl.when(sid + (it + 1) * NSUB < NCHUNK_M)
                def _():
                    wait_idx(it + 1, 1 - b)
                    issue_gather(1 - b)

                for j in range(CHM // 16):
                    sdid[b][0, pl.ds(16 * j, 16)] = (
                        eb[b][1, pl.ds(16 * j, 16)] - off)

                def group(g, gcarry):
                    acc = jnp.zeros((16,), jnp.float32)
                    for l in range(16):
                        e = 16 * g + l
                        t = None
                        for j in range(4):
                            h = jnp.maximum(
                                srows[b][e, pl.ds(16 * j, 16)]
                                + drows[b][e, pl.ds(16 * j, 16)], 0.0)
                            t = h * m2r[j] if t is None else t + h * m2r[j]
                        for p in perms:
                            t = t + jnp.take_along_axis(t, p, axis=0)
                        acc = jnp.where(iota == l, t, acc)
                    z = acc + bias
                    wv = 1.0 / (1.0 + jnp.exp(-z))
                    wbuf[b][pl.ds(16 * g, 16)] = wv
                    for l in range(16):
                        wpad[b][16 * g + l] = jnp.where(iota == 0, wv[l], 0.0)
                    return gcarry

                lax.fori_loop(0, CHM // 16, group, None)
                pltpu.async_copy(
                    wbuf[b], wout.at[cid, pl.ds(cbase(it), CHM)], osem[b])
                pltpu.async_copy(wpad[b], accD.at[sdid[b].at[0]], ssem[b],
                                 add=True)

                @pl.when(sid + (it + 2) * NSUB < NCHUNK_M)
                def _():
                    issue_idx(it + 2, b)

        return carry

    lax.fori_loop(0, (ITERS_M + 1) // 2, outer, None)
    drain_out(0)
    drain_out(1)
    drain_scatter(0)
    drain_scatter(1)
    plsc.subcore_barrier()
    pltpu.sync_copy(accD.at[pl.ds(r0, RPT)], deg.at[cid, pl.ds(r0, RPT)])


@functools.cache
def _get_mask():
    return pl.kernel(
        _mask_body,
        out_type=(jax.ShapeDtypeStruct((2, E), jnp.float32),
                  jax.ShapeDtypeStruct((2, N, 16), jnp.float32)),
        mesh=_mesh(),
        scratch_types=[
            pltpu.VMEM((2, CHM), jnp.int32),
            pltpu.VMEM((2, CHM), jnp.int32),
            pltpu.VMEM((1, CHM), jnp.int32),
            pltpu.VMEM((1, CHM), jnp.int32),
            pltpu.VMEM((CHM, 64), jnp.float32),
            pltpu.VMEM((CHM, 64), jnp.float32),
            pltpu.VMEM((CHM, 64), jnp.float32),
            pltpu.VMEM((CHM, 64), jnp.float32),
            pltpu.VMEM((CHM,), jnp.float32),
            pltpu.VMEM((CHM,), jnp.float32),
            pltpu.VMEM((CHM, 16), jnp.float32),
            pltpu.VMEM((CHM, 16), jnp.float32),
            pltpu.VMEM((2, AW), jnp.float32),
            pltpu.VMEM_SHARED((N, 16), jnp.float32),
        ] + [pltpu.SemaphoreType.DMA] * 10,
        compiler_params=_SC_PARAMS,
    )


def _layer_body(same, *args):
    if same:
        (tcr, w, eidx, zc64, aggc, aggb,
         eb0, eb1, wb0, wb1, sdid0, sdid1,
         rowsc0, rowsc1, mbufc0, mbufc1, mbufb0, mbufb1,
         accC, accB,
         isem0, isem1, gsem0, gsem1, g2sem0, g2sem1, ssem0, ssem1) = args
        rowsb0, rowsb1 = rowsc0, rowsc1
        tbr = tcr
    else:
        (tcr, tbr, w, eidx, zc64, aggc, aggb,
         eb0, eb1, wb0, wb1, sdid0, sdid1,
         rowsc0, rowsc1, rowsb0, rowsb1, mbufc0, mbufc1, mbufb0, mbufb1,
         accC, accB,
         isem0, isem1, gsem0, gsem1, g2sem0, g2sem1, ssem0, ssem1) = args
    eb = (eb0, eb1)
    wb = (wb0, wb1)
    sdid = (sdid0, sdid1)
    rows_c = (rowsc0, rowsc1)
    rows_b = (rowsb0, rowsb1)
    mbufc = (mbufc0, mbufc1)
    mbufb = (mbufb0, mbufb1)
    isem = (isem0, isem1)
    gsem = (gsem0, gsem1)
    g2sem = (g2sem0, g2sem1)
    ssem = (ssem0, ssem1)
    cid = lax.axis_index("c")
    sid = lax.axis_index("s")
    off = cid * N

    def cbase(it):
        return (sid + it * NSUB) * CHL

    def issue_idx(it, b):
        pltpu.async_copy(eidx.at[:, pl.ds(cbase(it), CHL)], eb[b], isem[b])
        pltpu.async_copy(w.at[:, pl.ds(cbase(it), CHL)], wb[b], isem[b])

    def wait_idx(it, b):
        pltpu.make_async_copy(
            eidx.at[:, pl.ds(cbase(it), CHL)], eb[b], isem[b]).wait()
        pltpu.make_async_copy(
            w.at[:, pl.ds(cbase(it), CHL)], wb[b], isem[b]).wait()

    def issue_gather(b):
        for j in range(CHL // 16):
            eb[b][0, pl.ds(16 * j, 16)] = eb[b][0, pl.ds(16 * j, 16)] + off
        pltpu.async_copy(tcr.at[eb[b].at[0]], rows_c[b], gsem[b])
        if not same:
            pltpu.async_copy(tbr.at[eb[b].at[0]], rows_b[b], g2sem[b])

    def wait_gather(b):
        pltpu.make_async_copy(tcr.at[eb[b].at[0]], rows_c[b], gsem[b]).wait()
        if not same:
            pltpu.make_async_copy(
                tbr.at[eb[b].at[0]], rows_b[b], g2sem[b]).wait()

    def issue_scatter(b):
        pltpu.async_copy(mbufc[b], accC.at[sdid[b].at[0]], ssem[b], add=True)
        pltpu.async_copy(mbufb[b], accB.at[sdid[b].at[0]], ssem[b], add=True)

    def drain_scatter(b):
        pltpu.make_async_copy(mbufc[b], accC.at[sdid[b].at[0]],
                              ssem[b]).wait()
        pltpu.make_async_copy(mbufb[b], accB.at[sdid[b].at[0]],
                              ssem[b]).wait()

    r0 = sid * RPT
    pltpu.sync_copy(zc64.at[pl.ds(r0, RPT)], accC.at[pl.ds(r0, RPT)])
    pltpu.sync_copy(zc64.at[pl.ds(r0, RPT)], accB.at[pl.ds(r0, RPT)])
    issue_idx(0, 0)
    issue_idx(1, 1)
    wait_idx(0, 0)
    issue_gather(0)
    plsc.subcore_barrier()

    def outer(it2, carry):
        for u in range(2):
            it = it2 * 2 + u
            b = u
            c = sid + it * NSUB

            @pl.when(c < NCHUNK_L)
            def _():
                @pl.when(it >= 2)
                def _():
                    drain_scatter(b)

                wait_gather(b)

                @pl.when(sid + (it + 1) * NSUB < NCHUNK_L)
                def _():
                    wait_idx(it + 1, 1 - b)
                    issue_gather(1 - b)

                for j in range(CHL // 16):
                    sdid[b][0, pl.ds(16 * j, 16)] = eb[b][1, pl.ds(16 * j, 16)]

                def group(g, gcarry):
                    wcv = wb[b][0, pl.ds(16 * g, 16)]
                    wbv = wb[b][1, pl.ds(16 * g, 16)]
                    for l in range(16):
                        e = 16 * g + l
                        wce = wcv[l]
                        wbe = wbv[l]
                        for j in range(4):
                            rc = rows_c[b][e, pl.ds(16 * j, 16)]
                            mbufc[b][e, pl.ds(16 * j, 16)] = rc * wce
                            rb = (rc if same
                                  else rows_b[b][e, pl.ds(16 * j, 16)])
                            mbufb[b][e, pl.ds(16 * j, 16)] = rb * wbe
                    return gcarry

                lax.fori_loop(0, CHL // 16, group, None)
                issue_scatter(b)

                @pl.when(sid + (it + 2) * NSUB < NCHUNK_L)
                def _():
                    issue_idx(it + 2, b)

        return carry

    lax.fori_loop(0, (ITERS_L + 1) // 2, outer, None)
    drain_scatter(0)
    drain_scatter(1)
    plsc.subcore_barrier()
    pltpu.sync_copy(accC.at[pl.ds(r0, RPT)], aggc.at[cid, pl.ds(r0, RPT)])
    pltpu.sync_copy(accB.at[pl.ds(r0, RPT)], aggb.at[cid, pl.ds(r0, RPT)])


@functools.cache
def _make_layer(same):
    scratch = [
        pltpu.VMEM((2, CHL), jnp.int32),
        pltpu.VMEM((2, CHL), jnp.int32),
        pltpu.VMEM((2, CHL), jnp.float32),
        pltpu.VMEM((2, CHL), jnp.float32),
        pltpu.VMEM((1, CHL), jnp.int32),
        pltpu.VMEM((1, CHL), jnp.int32),
        pltpu.VMEM((CHL, 64), jnp.float32),
        pltpu.VMEM((CHL, 64), jnp.float32),
    ]
    if not same:
        scratch += [pltpu.VMEM((CHL, 64), jnp.float32),
                    pltpu.VMEM((CHL, 64), jnp.float32)]
    scratch += [
        pltpu.VMEM((CHL, 64), jnp.float32),
        pltpu.VMEM((CHL, 64), jnp.float32),
        pltpu.VMEM((CHL, 64), jnp.float32),
        pltpu.VMEM((CHL, 64), jnp.float32),
        pltpu.VMEM_SHARED((N, 64), jnp.float32),
        pltpu.VMEM_SHARED((N, 64), jnp.float32),
    ] + [pltpu.SemaphoreType.DMA] * 8
    return pl.kernel(
        functools.partial(_layer_body, same),
        out_type=(jax.ShapeDtypeStruct((2, N, 64), jnp.float32),
                  jax.ShapeDtypeStruct((2, N, 64), jnp.float32)),
        mesh=_mesh(),
        scratch_types=scratch,
        compiler_params=_SC_PARAMS,
    )


_RB = 1000
_GRID = N // _RB


def _prep_body(x_ref, mc1_ref, mc1b_ref, mb1_ref, mb1b_ref,
               psrc_ref, pdst_ref, xs_ref):
    xb = x_ref[...]
    f32 = jnp.float32
    psrc_ref[0] = jnp.dot(xb, mc1_ref[:F], preferred_element_type=f32) + mc1b_ref[...]
    psrc_ref[1] = jnp.dot(xb, mb1_ref[:F], preferred_element_type=f32) + mb1b_ref[...]
    pdst_ref[0] = jnp.dot(xb, mc1_ref[F:], preferred_element_type=f32)
    pdst_ref[1] = jnp.dot(xb, mb1_ref[F:], preferred_element_type=f32)
    xs_ref[0] = xb[:, :64]
    xs_ref[1] = xb[:, 64:]


_prep = pl.pallas_call(
    _prep_body,
    grid=(_GRID,),
    in_specs=[
        pl.BlockSpec((_RB, F), lambda i: (i, 0)),
        pl.BlockSpec((2 * F, 64), lambda i: (0, 0)),
        pl.BlockSpec((1, 64), lambda i: (0, 0)),
        pl.BlockSpec((2 * F, 64), lambda i: (0, 0)),
        pl.BlockSpec((1, 64), lambda i: (0, 0)),
    ],
    out_specs=[pl.BlockSpec((2, _RB, 64), lambda i: (0, i, 0))] * 3,
    out_shape=[jax.ShapeDtypeStruct((2, N, 64), jnp.float32)] * 3,
)


def _post1_body(aggc_ref, aggb_ref, deg_ref, wc1_ref, bc1_ref, wb1_ref,
                bb1_ref, hc_ref, hb_ref):
    for k, (agg_ref, w_ref, b_ref, h_ref) in enumerate((
        (aggc_ref, wc1_ref, bc1_ref, hc_ref),
        (aggb_ref, wb1_ref, bb1_ref, hb_ref),
    )):
        msg = jnp.concatenate([agg_ref[0], agg_ref[1]], axis=1)
        deg = jnp.clip(deg_ref[k, :, 0:1], 1e-6, None)
        h = jnp.dot(msg / deg, w_ref[...], preferred_element_type=jnp.float32)
        h = jnp.maximum(h + b_ref[...], 0.0)
        h_ref[0] = h[:, :64]
        h_ref[1] = h[:, 64:]


_post1 = pl.pallas_call(
    _post1_body,
    grid=(_GRID,),
    in_specs=[
        pl.BlockSpec((2, _RB, 64), lambda i: (0, i, 0)),
        pl.BlockSpec((2, _RB, 64), lambda i: (0, i, 0)),
        pl.BlockSpec((2, _RB, 16), lambda i: (0, i, 0)),
        pl.BlockSpec((F, F), lambda i: (0, 0)),
        pl.BlockSpec((1, F), lambda i: (0, 0)),
        pl.BlockSpec((F, F), lambda i: (0, 0)),
        pl.BlockSpec((1, F), lambda i: (0, 0)),
    ],
    out_specs=[pl.BlockSpec((2, _RB, 64), lambda i: (0, i, 0))] * 2,
    out_shape=[jax.ShapeDtypeStruct((2, N, 64), jnp.float32)] * 2,
)


def _final_body(aggc_ref, aggb_ref, deg_ref, wc2_ref, bc2_ref, wb2_ref,
                bb2_ref, batch_ref, wdp_ref, wsab_ref, biasp_ref,
                zc_ref, zb_ref, heads_ref, accc, accb, cnt):
    i = pl.program_id(0)
    f32 = jnp.float32

    @pl.when(i == 0)
    def _():
        z = jnp.zeros((G, F), f32)
        accc[...] = z
        accb[...] = z
        cnt[...] = z

    hs = []
    for k, (agg_ref, w_ref, b_ref) in enumerate(((aggc_ref, wc2_ref, bc2_ref),
                                                 (aggb_ref, wb2_ref, bb2_ref))):
        msg = jnp.concatenate([agg_ref[0], agg_ref[1]], axis=1)
        deg = jnp.clip(deg_ref[k, :, 0:1], 1e-6, None)
        hs.append(jnp.dot(msg / deg, w_ref[...], preferred_element_type=f32)
                  + b_ref[...])
    h2c, h2b = hs
    bt = batch_ref[0, 0, :]
    oh = (bt[:, None]
          == lax.broadcasted_iota(jnp.int32, (_RB, G), 1)).astype(f32)
    dn = (((0,), (0,)), ((), ()))
    accc[...] += lax.dot_general(oh, h2c, dn, preferred_element_type=f32)
    accb[...] += lax.dot_general(oh, h2b, dn, preferred_element_type=f32)
    cnt[...] += lax.dot_general(oh, jnp.ones((_RB, F), f32), dn,
                                preferred_element_type=f32)

    @pl.when(i == pl.num_programs(0) - 1)
    def _():
        cc = jnp.clip(cnt[...], 1.0, None)
        zc = accc[...] / cc
        zb = accb[...] / cc
        zc_ref[...] = zc
        zb_ref[...] = zb
        zfc = jnp.concatenate([zc, 0.5 * zb], axis=1)
        zfb = jnp.concatenate([zc, zb], axis=1)
        heads_ref[...] = (
            jnp.dot(zfc, wdp_ref[...], preferred_element_type=f32)
            + jnp.dot(zfb, wsab_ref[...], preferred_element_type=f32)
            + biasp_ref[...])


_final = pl.pallas_call(
    _final_body,
    grid=(_GRID,),
    in_specs=[
        pl.BlockSpec((2, _RB, 64), lambda i: (0, i, 0)),
        pl.BlockSpec((2, _RB, 64), lambda i: (0, i, 0)),
        pl.BlockSpec((2, _RB, 16), lambda i: (0, i, 0)),
        pl.BlockSpec((F, F), lambda i: (0, 0)),
        pl.BlockSpec((1, F), lambda i: (0, 0)),
        pl.BlockSpec((F, F), lambda i: (0, 0)),
        pl.BlockSpec((1, F), lambda i: (0, 0)),
        pl.BlockSpec((1, 1, _RB), lambda i: (i, 0, 0)),
        pl.BlockSpec((2 * F, F), lambda i: (0, 0)),
        pl.BlockSpec((2 * F, F), lambda i: (0, 0)),
        pl.BlockSpec((1, F), lambda i: (0, 0)),
    ],
    out_specs=[pl.BlockSpec((G, F), lambda i: (0, 0))] * 3,
    out_shape=[jax.ShapeDtypeStruct((G, F), jnp.float32)] * 3,
    scratch_shapes=[pltpu.VMEM((G, F), jnp.float32)] * 3,
)


def kernel(x, edge_index, batch, Wc1, bc1, Wc2, bc2, Wb1, bb1, Wb2, bb2,
           Mc1, mc1, Mc2, mc2, Mb1, mb1, Mb2, mb2,
           Wd, bd, Ws, bs, Wa, ba, Wdb, bdb):
    f32 = jnp.float32
    psrcS, pdstS, xS = _prep(x, Mc1, mc1.reshape(1, 64), Mb1, mb1.reshape(1, 64))
    m2t = (jnp.zeros((2, AW), f32)
           .at[0, :64].set(Mc2[:, 0]).at[1, :64].set(Mb2[:, 0])
           .at[0, 64].set(mc2[0]).at[1, 64].set(mb2[0]))
    z16 = jnp.zeros((N, 16), f32)
    z64 = jnp.zeros((N, 64), f32)
    w2, deg = _get_mask()(psrcS.reshape(2 * N, 64), pdstS.reshape(2 * N, 64),
                          edge_index, m2t, z16)
    agg1c, agg1b = _make_layer(True)(xS.reshape(2 * N, 64), w2, edge_index,
                                     z64)
    h1cS, h1bS = _post1(agg1c, agg1b, deg, Wc1, bc1.reshape(1, F),
                        Wb1, bb1.reshape(1, F))
    agg2c, agg2b = _make_layer(False)(h1cS.reshape(2 * N, 64),
                                      h1bS.reshape(2 * N, 64), w2, edge_index,
                                      z64)
    WdP = jnp.zeros((2 * F, F), f32).at[:, :2].set(Wd)
    WsabP = (jnp.zeros((2 * F, F), f32)
             .at[:, 2:4].set(Ws).at[:, 4:8].set(Wa).at[:, 8:10].set(Wdb))
    biasP = (jnp.zeros((1, F), f32)
             .at[0, :2].set(bd).at[0, 2:4].set(bs)
             .at[0, 4:8].set(ba).at[0, 8:10].set(bdb))
    zc, zb, heads = _final(agg2c, agg2b, deg, Wc2, bc2.reshape(1, F),
                           Wb2, bb2.reshape(1, F),
                           batch.reshape(_GRID, 1, _RB), WdP, WsabP, biasP)
    return (heads[:, :2], heads[:, 2:4], heads[:, 4:8], heads[:, 8:10],
            zc, zb, zc, zb)
